# Initial kernel scaffold; baseline (speedup 1.0000x reference)
import functools

import jax
import jax.numpy as jnp
from jax import lax
from jax.experimental import pallas as pl
from jax.experimental.pallas import tpu as pltpu
from jax.experimental.pallas import tpu_sc as plsc

N = 50000
NPAD = 50048
NBLK = NPAD // 128
E = 300000
EPAD = 303104
NCHUNK = 148
ROWS_PER_TILE = NPAD // 16
ZROWS = ROWS_PER_TILE // 8
QROWS = ROWS_PER_TILE // 4

_mesh = plsc.VectorSubcoreMesh(core_axis_name="c", subcore_axis_name="s")


@jax.jit
def _deg_call(cols0, cols1):
    @functools.partial(
        pl.kernel,
        mesh=_mesh,
        out_type=[jax.ShapeDtypeStruct((NPAD, 16), jnp.float32)] * 2,
        scratch_types=[
            pltpu.VMEM((NCHUNK, 128), jnp.int32),
            pltpu.VMEM((128, 16), jnp.float32),
            pltpu.VMEM((QROWS, 16), jnp.float32),
            pltpu.VMEM((QROWS, 16), jnp.float32),
            pltpu.SemaphoreType.DMA,
        ],
    )
    def deg_kernel(c0_hbm, c1_hbm, d0_hbm, d1_hbm,
                   colv, ones_v, zb, bounce, sem):
        c = lax.axis_index("c")
        s = lax.axis_index("s")

        @pl.loop(0, 128)
        def _(i):
            ones_v[i] = jnp.full((16,), 1.0, jnp.float32)

        @pl.loop(0, QROWS)
        def _(i):
            zb[i] = jnp.zeros((16,), jnp.float32)

        def per_core(cols_hbm, deg_hbm):
            def body(accum):
                pltpu.sync_copy(cols_hbm.at[s], colv)
                base = s * ROWS_PER_TILE
                for k in range(4):
                    pltpu.sync_copy(zb, accum.at[pl.ds(base + k * QROWS, QROWS)])
                plsc.subcore_barrier()

                @pl.loop(0, NCHUNK)
                def _(j):
                    pltpu.sync_copy(ones_v, accum.at[colv.at[j]], add=True)

                plsc.subcore_barrier()
                for k in range(4):
                    sl = pl.ds(base + k * QROWS, QROWS)
                    pltpu.sync_copy(accum.at[sl], bounce)
                    pltpu.sync_copy(bounce, deg_hbm.at[sl])

            pl.run_scoped(
                body,
                accum=pltpu.VMEM_SHARED((NPAD, 16), jnp.float32),
            )

        @pl.when(c == 0)
        def _():
            per_core(c0_hbm, d0_hbm)

        @pl.when(c == 1)
        def _():
            per_core(c1_hbm, d1_hbm)

    return deg_kernel(cols0, cols1)


@jax.jit
def _agg_call(rows, cols, y4):
    @functools.partial(
        pl.kernel,
        mesh=_mesh,
        out_type=jax.ShapeDtypeStruct((4, NPAD, 32), jnp.float32),
        scratch_types=[
            pltpu.VMEM((NCHUNK, 128), jnp.int32),
            pltpu.VMEM((NCHUNK, 128), jnp.int32),
            pltpu.VMEM((128, 32), jnp.float32),
            pltpu.VMEM((128, 32), jnp.float32),
            pltpu.VMEM((128, 32), jnp.float32),
            pltpu.VMEM((128, 32), jnp.float32),
            pltpu.VMEM((ZROWS, 32), jnp.float32),
            pltpu.VMEM((ZROWS, 32), jnp.float32),
            pltpu.SemaphoreType.DMA,
            pltpu.SemaphoreType.DMA,
            pltpu.SemaphoreType.DMA,
            pltpu.SemaphoreType.DMA,
            pltpu.SemaphoreType.DMA,
            pltpu.SemaphoreType.DMA,
            pltpu.SemaphoreType.DMA,
            pltpu.SemaphoreType.DMA,
        ],
    )
    def agg_kernel(rows_hbm, cols_hbm, y4_hbm, agg_hbm,
                   rowv, colv, m0, m1, m2, m3, zb, bounce,
                   g0, g1, g2, g3, s0, s1, s2, s3):
        c = lax.axis_index("c")
        s = lax.axis_index("s")
        msgs = (m0, m1, m2, m3)
        gsems = (g0, g1, g2, g3)
        ssems = (s0, s1, s2, s3)

        pltpu.sync_copy(rows_hbm.at[s], rowv)
        pltpu.sync_copy(cols_hbm.at[s], colv)

        @pl.loop(0, ZROWS)
        def _(i):
            zb[i, pl.ds(0, 16)] = jnp.zeros((16,), jnp.float32)
            zb[i, pl.ds(16, 16)] = jnp.zeros((16,), jnp.float32)

        def body(accum):
            def one_slice(f):
                ysrc = y4_hbm.at[f]
                base = s * ROWS_PER_TILE
                for k in range(8):
                    pltpu.sync_copy(zb, accum.at[pl.ds(base + k * ZROWS, ZROWS)])
                plsc.subcore_barrier()

                for b in range(4):
                    pltpu.async_copy(ysrc.at[rowv.at[b]], msgs[b], gsems[b])

                @pl.loop(0, NCHUNK // 4)
                def _(j4):
                    j = j4 * 4
                    for b in range(4):
                        pltpu.make_async_copy(
                            ysrc.at[rowv.at[j + b]], msgs[b], gsems[b]).wait()
                        pltpu.async_copy(
                            msgs[b], accum.at[colv.at[j + b]], ssems[b],
                            add=True)

                    @pl.when(j4 < NCHUNK // 4 - 1)
                    def _():
                        for b in range(4):
                            pltpu.make_async_copy(
                                msgs[b], accum.at[colv.at[j + b]],
                                ssems[b]).wait()
                            pltpu.async_copy(
                                ysrc.at[rowv.at[j + 4 + b]], msgs[b], gsems[b])

                for b in range(4):
                    pltpu.make_async_copy(
                        msgs[b], accum.at[colv.at[NCHUNK - 4 + b]],
                        ssems[b]).wait()
                plsc.subcore_barrier()
                for k in range(8):
                    sl = pl.ds(base + k * ZROWS, ZROWS)
                    pltpu.sync_copy(accum.at[sl], bounce)
                    pltpu.sync_copy(bounce, agg_hbm.at[f].at[sl])

            @pl.when(c == 0)
            def _():
                one_slice(0)
                one_slice(1)

            @pl.when(c == 1)
            def _():
                one_slice(2)
                one_slice(3)

        pl.run_scoped(
            body,
            accum=pltpu.VMEM_SHARED((NPAD, 32), jnp.float32),
        )

    return agg_kernel(rows, cols, y4)


def _matmul_body(x_ref, w_ref, deg_ref, y_ref, *, from_slices):
    if from_slices:
        xb = jnp.concatenate([x_ref[g] for g in range(4)], axis=1)
    else:
        xb = x_ref[...]
    mm = jnp.dot(xb, w_ref[...], preferred_element_type=jnp.float32)
    dis = lax.rsqrt(deg_ref[...] + 1.0)
    for f in range(4):
        y_ref[f] = dis * mm[:, f * 32:(f + 1) * 32]


@jax.jit
def _matmul2d(x2d, w, deg):
    return pl.pallas_call(
        functools.partial(_matmul_body, from_slices=False),
        grid=(NBLK,),
        in_specs=[
            pl.BlockSpec((128, 128), lambda i: (i, 0)),
            pl.BlockSpec((128, 128), lambda i: (0, 0)),
            pl.BlockSpec((128, 1), lambda i: (i, 0)),
        ],
        out_specs=pl.BlockSpec((4, 128, 32), lambda i: (0, i, 0)),
        out_shape=jax.ShapeDtypeStruct((4, NPAD, 32), jnp.float32),
    )(x2d, w, deg)


@jax.jit
def _matmul4(x4, w, deg):
    return pl.pallas_call(
        functools.partial(_matmul_body, from_slices=True),
        grid=(NBLK,),
        in_specs=[
            pl.BlockSpec((4, 128, 32), lambda i: (0, i, 0)),
            pl.BlockSpec((128, 128), lambda i: (0, 0)),
            pl.BlockSpec((128, 1), lambda i: (i, 0)),
        ],
        out_specs=pl.BlockSpec((4, 128, 32), lambda i: (0, i, 0)),
        out_shape=jax.ShapeDtypeStruct((4, NPAD, 32), jnp.float32),
    )(x4, w, deg)


def _combine1_body(a0, y0, d0, a1, y1, d1, ba, bb, h_ref):
    dis0 = lax.rsqrt(d0[...] + 1.0)
    dis1 = lax.rsqrt(d1[...] + 1.0)
    h = (dis0 * (a0[...] + y0[...]) + dis1 * (a1[...] + y1[...])
         + ba[...] + bb[...])
    h_ref[...] = jnp.maximum(h, 0.0)


@jax.jit
def _combine1(a0, y0, d0, a1, y1, d1, ba, bb):
    spec4 = pl.BlockSpec((4, 128, 32), lambda i: (0, i, 0))
    dspec = pl.BlockSpec((128, 1), lambda i: (i, 0))
    bspec = pl.BlockSpec((4, 1, 32), lambda i: (0, 0, 0))
    return pl.pallas_call(
        _combine1_body,
        grid=(NBLK,),
        in_specs=[spec4, spec4, dspec, spec4, spec4, dspec, bspec, bspec],
        out_specs=spec4,
        out_shape=jax.ShapeDtypeStruct((4, NPAD, 32), jnp.float32),
    )(a0, y0, d0, a1, y1, d1, ba, bb)


def _combine2_body(a0, y0, d0, a1, y1, d1, ba, bb, o_ref):
    dis0 = lax.rsqrt(d0[...] + 1.0)
    dis1 = lax.rsqrt(d1[...] + 1.0)
    pieces = [dis0 * (a0[f] + y0[f]) + dis1 * (a1[f] + y1[f])
              for f in range(4)]
    o_ref[...] = jnp.concatenate(pieces, axis=1) + ba[...] + bb[...]


@jax.jit
def _combine2(a0, y0, d0, a1, y1, d1, ba, bb):
    spec4 = pl.BlockSpec((4, 128, 32), lambda i: (0, i, 0))
    dspec = pl.BlockSpec((128, 1), lambda i: (i, 0))
    bspec = pl.BlockSpec((1, 128), lambda i: (0, 0))
    return pl.pallas_call(
        _combine2_body,
        grid=(NBLK,),
        in_specs=[spec4, spec4, dspec, spec4, spec4, dspec, bspec, bspec],
        out_specs=pl.BlockSpec((128, 128), lambda i: (i, 0)),
        out_shape=jax.ShapeDtypeStruct((NPAD, 128), jnp.float32),
    )(a0, y0, d0, a1, y1, d1, ba, bb)


def _prep_edges(edge_index):
    pad = EPAD - E
    rows = jnp.concatenate(
        [edge_index[0], jnp.zeros((pad,), jnp.int32)]).reshape(16, NCHUNK, 128)
    cols = jnp.concatenate(
        [edge_index[1], jnp.full((pad,), N, jnp.int32)]).reshape(16, NCHUNK, 128)
    return rows, cols


def kernel(x, edge_index_rel0, edge_index_rel1,
           W1_0, b1_0, W1_1, b1_1, W2_0, b2_0, W2_1, b2_1):
    x_pad = jnp.pad(x, ((0, NPAD - N), (0, 0)))
    rows0, cols0 = _prep_edges(edge_index_rel0)
    rows1, cols1 = _prep_edges(edge_index_rel1)

    deg0_16, deg1_16 = _deg_call(cols0, cols1)
    deg0 = deg0_16[:, :1]
    deg1 = deg1_16[:, :1]

    y10 = _matmul2d(x_pad, W1_0, deg0)
    y11 = _matmul2d(x_pad, W1_1, deg1)
    a10 = _agg_call(rows0, cols0, y10)
    a11 = _agg_call(rows1, cols1, y11)
    h4 = _combine1(a10, y10, deg0, a11, y11, deg1,
                   b1_0.reshape(4, 1, 32), b1_1.reshape(4, 1, 32))

    y20 = _matmul4(h4, W2_0, deg0)
    y21 = _matmul4(h4, W2_1, deg1)
    a20 = _agg_call(rows0, cols0, y20)
    a21 = _agg_call(rows1, cols1, y21)
    out = _combine2(a20, y20, deg0, a21, y21, deg1,
                    b2_0.reshape(1, 128), b2_1.reshape(1, 128))
    return out[:N]


# speedup vs baseline: 4.8496x; 4.8496x over previous
import dataclasses
import functools

import jax
import jax.numpy as jnp
from jax import lax
from jax.experimental import pallas as pl
from jax.experimental.pallas import tpu as pltpu
from jax.experimental.pallas import tpu_sc as plsc

N = 50000
NPAD = 50048
NBLK = NPAD // 128
E = 300000
EPAD = 303104
NCHUNK = 148
NF = 8
FW = 128 // NF
ROWS_PER_TILE = NPAD // 16
ZROWS = 184
NZ = ROWS_PER_TILE // ZROWS
EPT = EPAD // 16

_mesh = plsc.VectorSubcoreMesh(core_axis_name="c", subcore_axis_name="s")

_sc_params = pltpu.CompilerParams()
if "needs_layout_passes" in pltpu.CompilerParams.__dataclass_fields__:
    _sc_params = dataclasses.replace(_sc_params, needs_layout_passes=False)
_sc_linear_params = _sc_params
if "use_tc_tiling_on_sc" in pltpu.CompilerParams.__dataclass_fields__:
    _sc_linear_params = dataclasses.replace(
        _sc_linear_params, use_tc_tiling_on_sc=False)


@jax.jit
def _deg_call(cols0, cols1):
    @functools.partial(
        pl.kernel,
        mesh=_mesh,
        out_type=[jax.ShapeDtypeStruct((16, NPAD), jnp.float32)] * 2,
        compiler_params=_sc_params,
        scratch_types=[
            pltpu.VMEM((EPT,), jnp.int32),
            pltpu.VMEM((NPAD,), jnp.float32),
        ],
    )
    def deg_kernel(c0_hbm, c1_hbm, p0_hbm, p1_hbm, colv, hist):
        c = lax.axis_index("c")
        s = lax.axis_index("s")

        @pl.loop(0, NPAD // 16)
        def _(i):
            hist[pl.ds(i * 16, 16)] = jnp.zeros((16,), jnp.float32)

        def per_core(cols_hbm, part_hbm):
            pltpu.sync_copy(cols_hbm.at[s], colv)

            @pl.loop(0, EPT // 16)
            def _(i):
                idx = colv[pl.ds(i * 16, 16)]
                plsc.addupdate_scatter(hist, [idx],
                                       jnp.ones((16,), jnp.float32))

            pltpu.sync_copy(hist, part_hbm.at[s])

        @pl.when(c == 0)
        def _():
            per_core(c0_hbm, p0_hbm)

        @pl.when(c == 1)
        def _():
            per_core(c1_hbm, p1_hbm)

    return deg_kernel(cols0, cols1)


def _degreduce_body(p0_ref, p1_ref, d0_ref, d1_ref):
    d0_ref[...] = lax.rsqrt(jnp.sum(p0_ref[...], axis=0, keepdims=True) + 1.0)
    d1_ref[...] = lax.rsqrt(jnp.sum(p1_ref[...], axis=0, keepdims=True) + 1.0)


@jax.jit
def _degreduce(p0, p1):
    pspec = pl.BlockSpec((16, 128), lambda i: (0, i))
    ospec = pl.BlockSpec((1, 128), lambda i: (0, i))
    return pl.pallas_call(
        _degreduce_body,
        grid=(NBLK,),
        in_specs=[pspec, pspec],
        out_specs=[ospec, ospec],
        out_shape=[jax.ShapeDtypeStruct((1, NPAD), jnp.float32)] * 2,
    )(p0, p1)


@jax.jit
def _agg_dual(rows0, cols0, y40, rows1, cols1, y41):
    @functools.partial(
        pl.kernel,
        mesh=_mesh,
        out_type=[jax.ShapeDtypeStruct((NF, NPAD, FW), jnp.float32)] * 2,
        compiler_params=_sc_linear_params,
        scratch_types=[
            pltpu.VMEM((NCHUNK, 128), jnp.int32),
            pltpu.VMEM((NCHUNK, 128), jnp.int32),
            pltpu.VMEM((128, FW), jnp.float32),
            pltpu.VMEM((128, FW), jnp.float32),
            pltpu.VMEM((128, FW), jnp.float32),
            pltpu.VMEM((128, FW), jnp.float32),
            pltpu.VMEM((ZROWS, FW), jnp.float32),
            pltpu.VMEM((ZROWS, FW), jnp.float32),
            pltpu.VMEM_SHARED((NPAD, FW), jnp.float32),
            pltpu.SemaphoreType.DMA,
            pltpu.SemaphoreType.DMA,
            pltpu.SemaphoreType.DMA,
            pltpu.SemaphoreType.DMA,
            pltpu.SemaphoreType.DMA,
            pltpu.SemaphoreType.DMA,
            pltpu.SemaphoreType.DMA,
            pltpu.SemaphoreType.DMA,
        ],
    )
    def agg_kernel(rows0_hbm, cols0_hbm, y40_hbm, rows1_hbm, cols1_hbm,
                   y41_hbm, agg0_hbm, agg1_hbm,
                   rowv, colv, m0, m1, m2, m3, zb, bounce, accum,
                   g0, g1, g2, g3, s0, s1, s2, s3):
        c = lax.axis_index("c")
        s = lax.axis_index("s")
        msgs = (m0, m1, m2, m3)
        gsems = (g0, g1, g2, g3)
        ssems = (s0, s1, s2, s3)

        @pl.loop(0, ZROWS)
        def _(i):
            zb[i, pl.ds(0, 16)] = jnp.zeros((16,), jnp.float32)

        def per_core(rows_hbm, cols_hbm, y4_hbm, agg_hbm):
            pltpu.sync_copy(rows_hbm.at[s], rowv)
            pltpu.sync_copy(cols_hbm.at[s], colv)
            base = s * ROWS_PER_TILE

            @pl.loop(0, NF)
            def _(f):
                ysrc = y4_hbm.at[f]

                @pl.loop(0, NZ)
                def _(k):
                    pltpu.sync_copy(zb, accum.at[pl.ds(base + k * ZROWS, ZROWS)])

                plsc.subcore_barrier()

                for b in range(4):
                    pltpu.async_copy(ysrc.at[rowv.at[b]], msgs[b], gsems[b])

                @pl.loop(0, NCHUNK // 4)
                def _(j4):
                    j = j4 * 4
                    for b in range(4):
                        pltpu.make_async_copy(
                            ysrc.at[rowv.at[j + b]], msgs[b], gsems[b]).wait()
                        pltpu.async_copy(
                            msgs[b], accum.at[colv.at[j + b]], ssems[b],
                            add=True)

                    @pl.when(j4 < NCHUNK // 4 - 1)
                    def _():
                        for b in range(4):
                            pltpu.make_async_copy(
                                msgs[b], accum.at[colv.at[j + b]],
                                ssems[b]).wait()
                            pltpu.async_copy(
                                ysrc.at[rowv.at[j + 4 + b]], msgs[b], gsems[b])

                for b in range(4):
                    pltpu.make_async_copy(
                        msgs[b], accum.at[colv.at[NCHUNK - 4 + b]],
                        ssems[b]).wait()
                plsc.subcore_barrier()

                @pl.loop(0, NZ)
                def _(k):
                    sl = pl.ds(base + k * ZROWS, ZROWS)
                    pltpu.sync_copy(accum.at[sl], bounce)
                    pltpu.sync_copy(bounce, agg_hbm.at[f].at[sl])

        @pl.when(c == 0)
        def _():
            per_core(rows0_hbm, cols0_hbm, y40_hbm, agg0_hbm)

        @pl.when(c == 1)
        def _():
            per_core(rows1_hbm, cols1_hbm, y41_hbm, agg1_hbm)

    return agg_kernel(rows0, cols0, y40, rows1, cols1, y41)


def _mm1_body(x_ref, w0_ref, w1_ref, d0_ref, d1_ref, y0_ref, y1_ref):
    xb = x_ref[...]
    mm0 = jnp.dot(xb, w0_ref[...], preferred_element_type=jnp.float32)
    mm1 = jnp.dot(xb, w1_ref[...], preferred_element_type=jnp.float32)
    dis0 = d0_ref[...]
    dis1 = d1_ref[...]
    for f in range(NF):
        y0_ref[f] = dis0 * mm0[:, f * FW:(f + 1) * FW]
        y1_ref[f] = dis1 * mm1[:, f * FW:(f + 1) * FW]


@jax.jit
def _mm1_dual(x2d, w0, w1, dis0, dis1):
    wspec = pl.BlockSpec((128, 128), lambda i: (0, 0))
    dspec = pl.BlockSpec((128, 1), lambda i: (i, 0))
    yspec = pl.BlockSpec((NF, 128, FW), lambda i: (0, i, 0))
    return pl.pallas_call(
        _mm1_body,
        grid=(NBLK,),
        in_specs=[pl.BlockSpec((128, 128), lambda i: (i, 0)),
                  wspec, wspec, dspec, dspec],
        out_specs=[yspec, yspec],
        out_shape=[jax.ShapeDtypeStruct((NF, NPAD, FW), jnp.float32)] * 2,
    )(x2d, w0, w1, dis0, dis1)


def _combmm_body(a0, y0, d0, a1, y1, d1, ba, bb, w0_ref, w1_ref,
                 y20_ref, y21_ref):
    dis0 = d0[...]
    dis1 = d1[...]
    h = (dis0 * (a0[...] + y0[...]) + dis1 * (a1[...] + y1[...])
         + ba[...] + bb[...])
    h = jnp.maximum(h, 0.0)
    hb = jnp.concatenate([h[g] for g in range(NF)], axis=1)
    mm0 = jnp.dot(hb, w0_ref[...], preferred_element_type=jnp.float32)
    mm1 = jnp.dot(hb, w1_ref[...], preferred_element_type=jnp.float32)
    for f in range(NF):
        y20_ref[f] = dis0 * mm0[:, f * FW:(f + 1) * FW]
        y21_ref[f] = dis1 * mm1[:, f * FW:(f + 1) * FW]


@jax.jit
def _combmm2(a0, y0, d0, a1, y1, d1, ba, bb, w20, w21):
    spec4 = pl.BlockSpec((NF, 128, FW), lambda i: (0, i, 0))
    dspec = pl.BlockSpec((128, 1), lambda i: (i, 0))
    bspec = pl.BlockSpec((NF, 1, FW), lambda i: (0, 0, 0))
    wspec = pl.BlockSpec((128, 128), lambda i: (0, 0))
    return pl.pallas_call(
        _combmm_body,
        grid=(NBLK,),
        in_specs=[spec4, spec4, dspec, spec4, spec4, dspec, bspec, bspec,
                  wspec, wspec],
        out_specs=[spec4, spec4],
        out_shape=[jax.ShapeDtypeStruct((NF, NPAD, FW), jnp.float32)] * 2,
    )(a0, y0, d0, a1, y1, d1, ba, bb, w20, w21)


def _combine2_body(a0, y0, d0, a1, y1, d1, ba, bb, o_ref):
    dis0 = d0[...]
    dis1 = d1[...]
    pieces = [dis0 * (a0[f] + y0[f]) + dis1 * (a1[f] + y1[f])
              for f in range(NF)]
    o_ref[...] = jnp.concatenate(pieces, axis=1) + ba[...] + bb[...]


@jax.jit
def _combine2(a0, y0, d0, a1, y1, d1, ba, bb):
    spec4 = pl.BlockSpec((NF, 128, FW), lambda i: (0, i, 0))
    dspec = pl.BlockSpec((128, 1), lambda i: (i, 0))
    bspec = pl.BlockSpec((1, 128), lambda i: (0, 0))
    return pl.pallas_call(
        _combine2_body,
        grid=(NBLK,),
        in_specs=[spec4, spec4, dspec, spec4, spec4, dspec, bspec, bspec],
        out_specs=pl.BlockSpec((128, 128), lambda i: (i, 0)),
        out_shape=jax.ShapeDtypeStruct((NPAD, 128), jnp.float32),
    )(a0, y0, d0, a1, y1, d1, ba, bb)


def _prep_edges(edge_index):
    pad = EPAD - E
    rows = jnp.concatenate(
        [edge_index[0], jnp.zeros((pad,), jnp.int32)]).reshape(16, NCHUNK, 128)
    cols = jnp.concatenate(
        [edge_index[1], jnp.full((pad,), N, jnp.int32)]).reshape(16, NCHUNK, 128)
    return rows, cols


def kernel(x, edge_index_rel0, edge_index_rel1,
           W1_0, b1_0, W1_1, b1_1, W2_0, b2_0, W2_1, b2_1):
    x_pad = jnp.pad(x, ((0, NPAD - N), (0, 0)))
    rows0, cols0 = _prep_edges(edge_index_rel0)
    rows1, cols1 = _prep_edges(edge_index_rel1)

    p0, p1 = _deg_call(cols0.reshape(16, EPT), cols1.reshape(16, EPT))
    dis0_row, dis1_row = _degreduce(p0, p1)
    dis0 = dis0_row.reshape(NPAD, 1)
    dis1 = dis1_row.reshape(NPAD, 1)

    y10, y11 = _mm1_dual(x_pad, W1_0, W1_1, dis0, dis1)
    a10, a11 = _agg_dual(rows0, cols0, y10, rows1, cols1, y11)
    y20, y21 = _combmm2(a10, y10, dis0, a11, y11, dis1,
                        b1_0.reshape(NF, 1, FW), b1_1.reshape(NF, 1, FW),
                        W2_0, W2_1)
    a20, a21 = _agg_dual(rows0, cols0, y20, rows1, cols1, y21)
    out = _combine2(a20, y20, dis0, a21, y21, dis1,
                    b2_0.reshape(1, 128), b2_1.reshape(1, 128))
    return out[:N]


# speedup vs baseline: 8.7728x; 1.8090x over previous
import dataclasses
import functools

import jax
import jax.numpy as jnp
from jax import lax
from jax.experimental import pallas as pl
from jax.experimental.pallas import tpu as pltpu
from jax.experimental.pallas import tpu_sc as plsc

N = 50000
NPAD = 50048
NBLK = NPAD // 128
E = 300000
EPAD = 303104
NCHUNK = 148
NF = 8
FW = 128 // NF
ROWS_PER_TILE = NPAD // 16
ZROWS = 184
NZ = ROWS_PER_TILE // ZROWS
EPT = EPAD // 16

_mesh = plsc.VectorSubcoreMesh(core_axis_name="c", subcore_axis_name="s")

_sc_params = pltpu.CompilerParams()
if "needs_layout_passes" in pltpu.CompilerParams.__dataclass_fields__:
    _sc_params = dataclasses.replace(_sc_params, needs_layout_passes=False)
_sc_linear_params = _sc_params
if "use_tc_tiling_on_sc" in pltpu.CompilerParams.__dataclass_fields__:
    _sc_linear_params = dataclasses.replace(
        _sc_linear_params, use_tc_tiling_on_sc=False)


@jax.jit
def _deg_call(cols0, cols1):
    @functools.partial(
        pl.kernel,
        mesh=_mesh,
        out_type=[jax.ShapeDtypeStruct((16, NPAD), jnp.float32)] * 2,
        compiler_params=_sc_params,
        scratch_types=[
            pltpu.VMEM((EPT,), jnp.int32),
            pltpu.VMEM((NPAD,), jnp.float32),
        ],
    )
    def deg_kernel(c0_hbm, c1_hbm, p0_hbm, p1_hbm, colv, hist):
        c = lax.axis_index("c")
        s = lax.axis_index("s")

        @pl.loop(0, NPAD // 16)
        def _(i):
            hist[pl.ds(i * 16, 16)] = jnp.zeros((16,), jnp.float32)

        def per_core(cols_hbm, part_hbm):
            pltpu.sync_copy(cols_hbm.at[s], colv)

            @pl.loop(0, EPT // 16)
            def _(i):
                idx = colv[pl.ds(i * 16, 16)]
                plsc.addupdate_scatter(hist, [idx],
                                       jnp.ones((16,), jnp.float32))

            pltpu.sync_copy(hist, part_hbm.at[s])

        @pl.when(c == 0)
        def _():
            per_core(c0_hbm, p0_hbm)

        @pl.when(c == 1)
        def _():
            per_core(c1_hbm, p1_hbm)

    return deg_kernel(cols0, cols1)


def _degreduce_body(p0_ref, p1_ref, d0_ref, d1_ref):
    d0_ref[...] = lax.rsqrt(jnp.sum(p0_ref[...], axis=0, keepdims=True) + 1.0)
    d1_ref[...] = lax.rsqrt(jnp.sum(p1_ref[...], axis=0, keepdims=True) + 1.0)


@jax.jit
def _degreduce(p0, p1):
    pspec = pl.BlockSpec((16, 2944), lambda i: (0, i))
    ospec = pl.BlockSpec((1, 2944), lambda i: (0, i))
    return pl.pallas_call(
        _degreduce_body,
        grid=(NPAD // 2944,),
        in_specs=[pspec, pspec],
        out_specs=[ospec, ospec],
        out_shape=[jax.ShapeDtypeStruct((1, NPAD), jnp.float32)] * 2,
    )(p0, p1)


@jax.jit
def _agg_dual(rows0, cols0, y40, rows1, cols1, y41):
    @functools.partial(
        pl.kernel,
        mesh=_mesh,
        out_type=[jax.ShapeDtypeStruct((NPAD, NF, FW), jnp.float32)] * 2,
        compiler_params=_sc_linear_params,
        scratch_types=[
            pltpu.VMEM((NCHUNK, 128), jnp.int32),
            pltpu.VMEM((NCHUNK, 128), jnp.int32),
            pltpu.VMEM((NCHUNK, 128), jnp.int32),
            pltpu.VMEM((128, FW), jnp.float32),
            pltpu.VMEM((128, FW), jnp.float32),
            pltpu.VMEM((128, FW), jnp.float32),
            pltpu.VMEM((128, FW), jnp.float32),
            pltpu.VMEM((ZROWS, FW), jnp.float32),
            pltpu.VMEM((ZROWS, FW), jnp.float32),
            pltpu.VMEM_SHARED((NPAD, FW), jnp.float32),
            pltpu.SemaphoreType.DMA,
            pltpu.SemaphoreType.DMA,
            pltpu.SemaphoreType.DMA,
            pltpu.SemaphoreType.DMA,
            pltpu.SemaphoreType.DMA,
            pltpu.SemaphoreType.DMA,
            pltpu.SemaphoreType.DMA,
            pltpu.SemaphoreType.DMA,
        ],
    )
    def agg_kernel(rows0_hbm, cols0_hbm, y40_hbm, rows1_hbm, cols1_hbm,
                   y41_hbm, agg0_hbm, agg1_hbm,
                   rowv, idxf, colv, m0, m1, m2, m3, zb, bounce, accum,
                   g0, g1, g2, g3, s0, s1, s2, s3):
        c = lax.axis_index("c")
        s = lax.axis_index("s")
        msgs = (m0, m1, m2, m3)
        gsems = (g0, g1, g2, g3)
        ssems = (s0, s1, s2, s3)

        @pl.loop(0, ZROWS)
        def _(i):
            zb[i, pl.ds(0, 16)] = jnp.zeros((16,), jnp.float32)

        def per_core(rows_hbm, cols_hbm, y4_hbm, agg_hbm):
            pltpu.sync_copy(rows_hbm.at[s], rowv)
            pltpu.sync_copy(cols_hbm.at[s], colv)
            base = s * ROWS_PER_TILE

            @pl.loop(0, NCHUNK)
            def _(j):
                @pl.loop(0, 8)
                def _(k):
                    sl = pl.ds(k * 16, 16)
                    rowv[j, sl] = rowv[j, sl] * NF

            @pl.loop(0, NF)
            def _(f):
                ysrc = y4_hbm

                @pl.loop(0, NCHUNK)
                def _(j):
                    @pl.loop(0, 8)
                    def _(k):
                        sl = pl.ds(k * 16, 16)
                        idxf[j, sl] = rowv[j, sl] + f

                @pl.loop(0, NZ)
                def _(k):
                    pltpu.sync_copy(zb, accum.at[pl.ds(base + k * ZROWS, ZROWS)])

                plsc.subcore_barrier()

                for b in range(4):
                    pltpu.async_copy(ysrc.at[idxf.at[b]], msgs[b], gsems[b])

                @pl.loop(0, NCHUNK // 4)
                def _(j4):
                    j = j4 * 4
                    for b in range(4):
                        pltpu.make_async_copy(
                            ysrc.at[idxf.at[j + b]], msgs[b], gsems[b]).wait()
                        pltpu.async_copy(
                            msgs[b], accum.at[colv.at[j + b]], ssems[b],
                            add=True)

                    @pl.when(j4 < NCHUNK // 4 - 1)
                    def _():
                        for b in range(4):
                            pltpu.make_async_copy(
                                msgs[b], accum.at[colv.at[j + b]],
                                ssems[b]).wait()
                            pltpu.async_copy(
                                ysrc.at[idxf.at[j + 4 + b]], msgs[b], gsems[b])

                for b in range(4):
                    pltpu.make_async_copy(
                        msgs[b], accum.at[colv.at[NCHUNK - 4 + b]],
                        ssems[b]).wait()
                plsc.subcore_barrier()

                @pl.loop(0, NZ)
                def _(k):
                    sl = pl.ds(base + k * ZROWS, ZROWS)
                    pltpu.sync_copy(accum.at[sl], bounce)
                    pltpu.sync_copy(bounce, agg_hbm.at[sl, f])

        @pl.when(c == 0)
        def _():
            per_core(rows0_hbm, cols0_hbm, y40_hbm, agg0_hbm)

        @pl.when(c == 1)
        def _():
            per_core(rows1_hbm, cols1_hbm, y41_hbm, agg1_hbm)

    return agg_kernel(rows0, cols0, y40, rows1, cols1, y41)


RB = 2944
NRB = NPAD // RB


def _mm1_body(x_ref, w0_ref, w1_ref, d0_ref, d1_ref, y0_ref, y1_ref):
    xb = x_ref[...]
    y0_ref[...] = d0_ref[...] * jnp.dot(
        xb, w0_ref[...], preferred_element_type=jnp.float32)
    y1_ref[...] = d1_ref[...] * jnp.dot(
        xb, w1_ref[...], preferred_element_type=jnp.float32)


@jax.jit
def _mm1_dual(x2d, w0, w1, dis0, dis1):
    wspec = pl.BlockSpec((128, 128), lambda i: (0, 0))
    dspec = pl.BlockSpec((RB, 1), lambda i: (i, 0))
    nspec = pl.BlockSpec((RB, 128), lambda i: (i, 0))
    return pl.pallas_call(
        _mm1_body,
        grid=(NRB,),
        in_specs=[nspec, wspec, wspec, dspec, dspec],
        out_specs=[nspec, nspec],
        out_shape=[jax.ShapeDtypeStruct((NPAD, 128), jnp.float32)] * 2,
    )(x2d, w0, w1, dis0, dis1)


def _combmm_body(a0, y0, d0, a1, y1, d1, ba, bb, w0_ref, w1_ref,
                 y20_ref, y21_ref):
    dis0 = d0[...]
    dis1 = d1[...]
    h = (dis0 * (a0[...] + y0[...]) + dis1 * (a1[...] + y1[...])
         + ba[...] + bb[...])
    h = jnp.maximum(h, 0.0)
    y20_ref[...] = dis0 * jnp.dot(
        h, w0_ref[...], preferred_element_type=jnp.float32)
    y21_ref[...] = dis1 * jnp.dot(
        h, w1_ref[...], preferred_element_type=jnp.float32)


@jax.jit
def _combmm2(a0, y0, d0, a1, y1, d1, ba, bb, w20, w21):
    nspec = pl.BlockSpec((RB, 128), lambda i: (i, 0))
    dspec = pl.BlockSpec((RB, 1), lambda i: (i, 0))
    bspec = pl.BlockSpec((1, 128), lambda i: (0, 0))
    wspec = pl.BlockSpec((128, 128), lambda i: (0, 0))
    return pl.pallas_call(
        _combmm_body,
        grid=(NRB,),
        in_specs=[nspec, nspec, dspec, nspec, nspec, dspec, bspec, bspec,
                  wspec, wspec],
        out_specs=[nspec, nspec],
        out_shape=[jax.ShapeDtypeStruct((NPAD, 128), jnp.float32)] * 2,
    )(a0, y0, d0, a1, y1, d1, ba, bb, w20, w21)


def _combine2_body(a0, y0, d0, a1, y1, d1, ba, bb, o_ref):
    o_ref[...] = (d0[...] * (a0[...] + y0[...])
                  + d1[...] * (a1[...] + y1[...]) + ba[...] + bb[...])


@jax.jit
def _combine2(a0, y0, d0, a1, y1, d1, ba, bb):
    nspec = pl.BlockSpec((RB, 128), lambda i: (i, 0))
    dspec = pl.BlockSpec((RB, 1), lambda i: (i, 0))
    bspec = pl.BlockSpec((1, 128), lambda i: (0, 0))
    return pl.pallas_call(
        _combine2_body,
        grid=(NRB,),
        in_specs=[nspec, nspec, dspec, nspec, nspec, dspec, bspec, bspec],
        out_specs=nspec,
        out_shape=jax.ShapeDtypeStruct((NPAD, 128), jnp.float32),
    )(a0, y0, d0, a1, y1, d1, ba, bb)


def _prep_edges(edge_index):
    pad = EPAD - E
    rows = jnp.pad(edge_index[0], (0, pad)).reshape(16, NCHUNK, 128)
    cols = jnp.pad(edge_index[1], (0, pad),
                   constant_values=N).reshape(16, NCHUNK, 128)
    return rows, cols


def kernel(x, edge_index_rel0, edge_index_rel1,
           W1_0, b1_0, W1_1, b1_1, W2_0, b2_0, W2_1, b2_1):
    x_pad = jnp.pad(x, ((0, NPAD - N), (0, 0)))
    rows0, cols0 = _prep_edges(edge_index_rel0)
    rows1, cols1 = _prep_edges(edge_index_rel1)

    p0, p1 = _deg_call(cols0.reshape(16, EPT), cols1.reshape(16, EPT))
    dis0_row, dis1_row = _degreduce(p0, p1)
    dis0 = dis0_row.reshape(NPAD, 1)
    dis1 = dis1_row.reshape(NPAD, 1)

    y10, y11 = _mm1_dual(x_pad, W1_0, W1_1, dis0, dis1)
    a10, a11 = _agg_dual(rows0, cols0, y10.reshape(NPAD * NF, FW),
                         rows1, cols1, y11.reshape(NPAD * NF, FW))
    a10 = a10.reshape(NPAD, 128)
    a11 = a11.reshape(NPAD, 128)
    y20, y21 = _combmm2(a10, y10, dis0, a11, y11, dis1,
                        b1_0.reshape(1, 128), b1_1.reshape(1, 128),
                        W2_0, W2_1)
    a20, a21 = _agg_dual(rows0, cols0, y20.reshape(NPAD * NF, FW),
                         rows1, cols1, y21.reshape(NPAD * NF, FW))
    a20 = a20.reshape(NPAD, 128)
    a21 = a21.reshape(NPAD, 128)
    out = _combine2(a20, y20, dis0, a21, y21, dis1,
                    b2_0.reshape(1, 128), b2_1.reshape(1, 128))
    return out[:N]
